# Initial kernel scaffold; baseline (speedup 1.0000x reference)
import jax
import jax.numpy as jnp
from jax.experimental import pallas as pl


def kernel(x, edge_index, edge_weight, img, W1, b1, W2, b2, Wi1, bi1, Wi2, bi2):
    raise NotImplementedError("write your pallas kernel here")


# speedup vs baseline: 1.4937x; 1.4937x over previous
import functools

import jax
import jax.numpy as jnp
from jax import lax
from jax.experimental import pallas as pl
from jax.experimental.pallas import tpu as pltpu
from jax.experimental.pallas import tpu_sc as plsc

N = 10000
NPAD = 10240
E = 160000
EPAD = 163840
NUM_RANGES = 4
RROWS = NPAD // NUM_RANGES
F32 = jnp.float32
I32 = jnp.int32

_MESH = plsc.VectorSubcoreMesh(core_axis_name="c", subcore_axis_name="s")
_SC_PARAMS = pltpu.CompilerParams(needs_layout_passes=False)


def _iota16():
    return lax.iota(I32, 16)


def _deg_body(dst_hbm, ew_hbm, deg_hbm, dacc, dloc, ewloc):
    c = lax.axis_index("c")
    s = lax.axis_index("s")
    w = c * 16 + s
    z16 = jnp.zeros((16,), F32)

    def _zb(i, _):
        dacc[pl.ds(i * 16, 16)] = z16
        return 0
    lax.fori_loop(0, NPAD // 16, _zb, 0)

    base = w * 5120
    pltpu.sync_copy(dst_hbm.at[pl.ds(base, 5120)], dloc)
    pltpu.sync_copy(ew_hbm.at[pl.ds(base, 5120)], ewloc)

    def _eb(i, _):
        idx = dloc[pl.ds(i * 16, 16)]
        plsc.addupdate_scatter(dacc, [idx], ewloc[pl.ds(i * 16, 16)])
        return 0
    lax.fori_loop(0, 320, _eb, 0)
    pltpu.sync_copy(dacc, deg_hbm.at[pl.ds(w * NPAD, NPAD)])


_deg_kernel = pl.kernel(
    _deg_body,
    out_type=jax.ShapeDtypeStruct((32 * NPAD,), F32),
    mesh=_MESH,
    scratch_types=[
        pltpu.VMEM((NPAD,), F32),
        pltpu.VMEM((5120,), I32),
        pltpu.VMEM((5120,), F32),
    ],
    compiler_params=_SC_PARAMS,
)


def _dinv_body(deg_ref, out_ref):
    d = deg_ref[pl.ds(0, NPAD // 128), :]
    for k in range(1, 32):
        d = d + deg_ref[pl.ds(k * (NPAD // 128), NPAD // 128), :]
    out_ref[...] = lax.rsqrt(jnp.maximum(d, 1e-6))


def _dinv_tc(deg_parts):
    return pl.pallas_call(
        _dinv_body,
        out_shape=jax.ShapeDtypeStruct((NPAD // 128, 128), F32),
    )(deg_parts.reshape(32 * (NPAD // 128), 128)).reshape(NPAD)


def _norm_body(src_hbm, dst_hbm, ew_hbm, dinv_hbm, norm_hbm,
               dfull, sloc, dloc, ewloc, nout):
    c = lax.axis_index("c")
    s = lax.axis_index("s")
    w = c * 16 + s

    pltpu.sync_copy(dinv_hbm, dfull)
    base = w * 5120
    pltpu.sync_copy(src_hbm.at[pl.ds(base, 5120)], sloc)
    pltpu.sync_copy(dst_hbm.at[pl.ds(base, 5120)], dloc)
    pltpu.sync_copy(ew_hbm.at[pl.ds(base, 5120)], ewloc)

    def _nb(i, _):
        s16 = sloc[pl.ds(i * 16, 16)]
        d16 = dloc[pl.ds(i * 16, 16)]
        dv_s = plsc.load_gather(dfull, [s16])
        dv_d = plsc.load_gather(dfull, [d16])
        nout[pl.ds(i * 16, 16)] = ewloc[pl.ds(i * 16, 16)] * dv_s * dv_d
        return 0
    lax.fori_loop(0, 320, _nb, 0)
    pltpu.sync_copy(nout, norm_hbm.at[pl.ds(base, 5120)])


_norm_kernel = pl.kernel(
    _norm_body,
    out_type=jax.ShapeDtypeStruct((EPAD,), F32),
    mesh=_MESH,
    scratch_types=[
        pltpu.VMEM((NPAD,), F32),
        pltpu.VMEM((5120,), I32),
        pltpu.VMEM((5120,), I32),
        pltpu.VMEM((5120,), F32),
        pltpu.VMEM((5120,), F32),
    ],
    compiler_params=_SC_PARAMS,
)


def _spmm_body(W, src_hbm, dst_hbm, norm_hbm, x_hbm, out_hbm,
               acc, sbuf, dbuf, nbuf, csrc, coff, cnrm, rbuf):
    c = lax.axis_index("c")
    s = lax.axis_index("s")
    w = c * 16 + s
    z16 = jnp.zeros((16,), F32)
    z16i = jnp.zeros((16,), I32)
    wlanes = W // 16

    def _pass(p, _):
        rbase = p * 5120 + w * 160

        def _zb(i, _):
            acc[pl.ds(i * 16, 16)] = z16
            return 0
        lax.fori_loop(0, 160 * wlanes, _zb, 0)

        def _batch(b, _):
            bbase = b * 2048
            pltpu.sync_copy(src_hbm.at[pl.ds(bbase, 2048)], sbuf)
            pltpu.sync_copy(dst_hbm.at[pl.ds(bbase, 2048)], dbuf)
            pltpu.sync_copy(norm_hbm.at[pl.ds(bbase, 2048)], nbuf)

            def _cb(i, off):
                d16 = dbuf[pl.ds(i * 16, 16)]
                m = (d16 >= rbase) & (d16 < rbase + 160)
                plsc.store_compressed(csrc.at[pl.ds(off, 16)],
                                      sbuf[pl.ds(i * 16, 16)], mask=m)
                plsc.store_compressed(coff.at[pl.ds(off, 16)], d16 - rbase,
                                      mask=m)
                plsc.store_compressed(cnrm.at[pl.ds(off, 16)],
                                      nbuf[pl.ds(i * 16, 16)], mask=m)
                return off + jnp.sum(m.astype(I32))
            cnt = lax.fori_loop(0, 128, _cb, jnp.int32(0))

            csrc[pl.ds(cnt, 16)] = z16i
            coff[pl.ds(cnt, 16)] = z16i
            cnrm[pl.ds(cnt, 16)] = z16
            nch = (cnt + 15) // 16

            def _gb(ch, _):
                i16 = csrc[pl.ds(ch * 16, 16)]
                pltpu.sync_copy(x_hbm.at[i16], rbuf)
                n16 = cnrm[pl.ds(ch * 16, 16)]
                o16 = coff[pl.ds(ch * 16, 16)]
                for r in range(16):
                    n = n16[r]
                    o = o16[r] * W

                    def _jb(j, _):
                        q = o + j * 16
                        acc[pl.ds(q, 16)] = (acc[pl.ds(q, 16)]
                                             + n * rbuf[r, pl.ds(j * 16, 16)])
                        return 0
                    lax.fori_loop(0, wlanes, _jb, 0)
                return 0
            lax.fori_loop(0, nch, _gb, 0)
            return 0
        lax.fori_loop(0, 80, _batch, 0)

        pltpu.sync_copy(acc, out_hbm.at[pl.ds(rbase * W, 160 * W)])
        return 0
    lax.fori_loop(0, 2, _pass, 0)


def _make_spmm(W):
    return pl.kernel(
        functools.partial(_spmm_body, W),
        out_type=jax.ShapeDtypeStruct((NPAD * W,), F32),
        mesh=_MESH,
        scratch_types=[
            pltpu.VMEM((160 * W,), F32),
            pltpu.VMEM((2048,), I32),
            pltpu.VMEM((2048,), I32),
            pltpu.VMEM((2048,), F32),
            pltpu.VMEM((2176,), I32),
            pltpu.VMEM((2176,), I32),
            pltpu.VMEM((2176,), F32),
            pltpu.VMEM((16, W), F32),
        ],
        compiler_params=_SC_PARAMS,
    )


_spmm640 = _make_spmm(640)
_spmm512 = _make_spmm(512)


def _gcn_mm_body(t_ref, w1_ref, w2_ref, b2_ref, out_ref):
    h = jnp.dot(t_ref[...], w1_ref[...], preferred_element_type=F32)
    h = jnp.maximum(h, 0.0)
    out_ref[...] = (jnp.dot(h, w2_ref[...], preferred_element_type=F32)
                    + b2_ref[...])


def _gcn_mm(t_aug, w1_aug, w2, b2):
    return pl.pallas_call(
        _gcn_mm_body,
        grid=(NPAD // 256,),
        in_specs=[
            pl.BlockSpec((256, 640), lambda i: (i, 0)),
            pl.BlockSpec((640, 4096), lambda i: (0, 0)),
            pl.BlockSpec((4096, 512), lambda i: (0, 0)),
            pl.BlockSpec((1, 512), lambda i: (0, 0)),
        ],
        out_specs=pl.BlockSpec((256, 512), lambda i: (i, 0)),
        out_shape=jax.ShapeDtypeStruct((NPAD, 512), F32),
    )(t_aug, w1_aug, w2, b2)


def _score_body(img_ref, wi1_ref, bi1_ref, wi2_ref, bi2_ref, emb_ref,
                out_ref, feats_scr):
    @pl.when(pl.program_id(0) == 0)
    def _():
        f = jnp.dot(img_ref[...], wi1_ref[...], preferred_element_type=F32)
        f = jnp.maximum(f + bi1_ref[...], 0.0)
        feats_scr[...] = (jnp.dot(f, wi2_ref[...], preferred_element_type=F32)
                          + bi2_ref[...])

    out_ref[...] = lax.dot_general(
        feats_scr[...], emb_ref[...],
        dimension_numbers=(((1,), (1,)), ((), ())),
        preferred_element_type=F32)


def _score_mm(img, wi1, bi1, wi2, bi2, emb_pairs):
    return pl.pallas_call(
        _score_body,
        grid=(9216 // 256,),
        in_specs=[
            pl.BlockSpec((1024, 512), lambda i: (0, 0)),
            pl.BlockSpec((512, 768), lambda i: (0, 0)),
            pl.BlockSpec((1, 768), lambda i: (0, 0)),
            pl.BlockSpec((768, 512), lambda i: (0, 0)),
            pl.BlockSpec((1, 512), lambda i: (0, 0)),
            pl.BlockSpec((256, 512), lambda i: (i, 0)),
        ],
        out_specs=pl.BlockSpec((1024, 256), lambda i: (0, i)),
        out_shape=jax.ShapeDtypeStruct((1024, 9216), F32),
        scratch_shapes=[pltpu.VMEM((1024, 512), F32)],
    )(img, wi1, bi1, wi2, bi2, emb_pairs)


def kernel(x, edge_index, edge_weight, img, W1, b1, W2, b2,
           Wi1, bi1, Wi2, bi2):
    src = edge_index[0].astype(I32)
    dst = edge_index[1].astype(I32)
    ew = edge_weight.astype(F32)
    pad = EPAD - E
    src = jnp.pad(src, (0, pad))
    dst = jnp.pad(dst, (0, pad))
    ew = jnp.pad(ew, (0, pad))

    deg_parts = _deg_kernel(dst, ew)
    dinv = _dinv_tc(deg_parts)
    norm = _norm_kernel(src, dst, ew, dinv)

    n = x.shape[0]
    x_aug = jnp.concatenate(
        [x, jnp.ones((n, 1), F32), jnp.zeros((n, 39), F32)], axis=1)
    t_aug = _spmm640(src, dst, norm, x_aug).reshape(NPAD, 640)

    w1_aug = jnp.concatenate(
        [W1, b1[None, :], jnp.zeros((39, 4096), F32)], axis=0)
    g = _gcn_mm(t_aug, w1_aug, W2, b2[None, :])

    emb = _spmm512(src, dst, norm, g).reshape(NPAD, 512)
    emb_pairs = emb[1000:10216]
    score = _score_mm(img, Wi1, bi1[None, :], Wi2, bi2[None, :], emb_pairs)
    return score[:, :9000]


# speedup vs baseline: 1.5701x; 1.0512x over previous
import functools

import jax
import jax.numpy as jnp
from jax import lax
from jax.experimental import pallas as pl
from jax.experimental.pallas import tpu as pltpu
from jax.experimental.pallas import tpu_sc as plsc

N = 10000
NPAD = 10240
E = 160000
EPAD = 163840
NUM_RANGES = 4
RROWS = NPAD // NUM_RANGES
F32 = jnp.float32
I32 = jnp.int32

_MESH = plsc.VectorSubcoreMesh(core_axis_name="c", subcore_axis_name="s")
_SC_PARAMS = pltpu.CompilerParams(needs_layout_passes=False)


def _iota16():
    return lax.iota(I32, 16)


def _deg_body(dst_hbm, ew_hbm, deg_hbm, dacc, dloc, ewloc):
    c = lax.axis_index("c")
    s = lax.axis_index("s")
    w = c * 16 + s
    z16 = jnp.zeros((16,), F32)

    def _zb(i, _):
        dacc[pl.ds(i * 16, 16)] = z16
        return 0
    lax.fori_loop(0, NPAD // 16, _zb, 0)

    base = w * 5120
    pltpu.sync_copy(dst_hbm.at[pl.ds(base, 5120)], dloc)
    pltpu.sync_copy(ew_hbm.at[pl.ds(base, 5120)], ewloc)

    def _eb(i, _):
        idx = dloc[pl.ds(i * 16, 16)]
        plsc.addupdate_scatter(dacc, [idx], ewloc[pl.ds(i * 16, 16)])
        return 0
    lax.fori_loop(0, 320, _eb, 0)
    pltpu.sync_copy(dacc, deg_hbm.at[pl.ds(w * NPAD, NPAD)])


_deg_kernel = pl.kernel(
    _deg_body,
    out_type=jax.ShapeDtypeStruct((32 * NPAD,), F32),
    mesh=_MESH,
    scratch_types=[
        pltpu.VMEM((NPAD,), F32),
        pltpu.VMEM((5120,), I32),
        pltpu.VMEM((5120,), F32),
    ],
    compiler_params=_SC_PARAMS,
)


def _dinv_body(deg_ref, out_ref):
    d = deg_ref[pl.ds(0, NPAD // 128), :]
    for k in range(1, 32):
        d = d + deg_ref[pl.ds(k * (NPAD // 128), NPAD // 128), :]
    out_ref[...] = lax.rsqrt(jnp.maximum(d, 1e-6))


def _dinv_tc(deg_parts):
    return pl.pallas_call(
        _dinv_body,
        out_shape=jax.ShapeDtypeStruct((NPAD // 128, 128), F32),
    )(deg_parts.reshape(32 * (NPAD // 128), 128)).reshape(NPAD)


def _norm_body(src_hbm, dst_hbm, ew_hbm, dinv_hbm, norm_hbm,
               dfull, sloc, dloc, ewloc, nout):
    c = lax.axis_index("c")
    s = lax.axis_index("s")
    w = c * 16 + s

    pltpu.sync_copy(dinv_hbm, dfull)
    base = w * 5120
    pltpu.sync_copy(src_hbm.at[pl.ds(base, 5120)], sloc)
    pltpu.sync_copy(dst_hbm.at[pl.ds(base, 5120)], dloc)
    pltpu.sync_copy(ew_hbm.at[pl.ds(base, 5120)], ewloc)

    def _nb(i, _):
        s16 = sloc[pl.ds(i * 16, 16)]
        d16 = dloc[pl.ds(i * 16, 16)]
        dv_s = plsc.load_gather(dfull, [s16])
        dv_d = plsc.load_gather(dfull, [d16])
        nout[pl.ds(i * 16, 16)] = ewloc[pl.ds(i * 16, 16)] * dv_s * dv_d
        return 0
    lax.fori_loop(0, 320, _nb, 0)
    pltpu.sync_copy(nout, norm_hbm.at[pl.ds(base, 5120)])


_norm_kernel = pl.kernel(
    _norm_body,
    out_type=jax.ShapeDtypeStruct((EPAD,), F32),
    mesh=_MESH,
    scratch_types=[
        pltpu.VMEM((NPAD,), F32),
        pltpu.VMEM((5120,), I32),
        pltpu.VMEM((5120,), I32),
        pltpu.VMEM((5120,), F32),
        pltpu.VMEM((5120,), F32),
    ],
    compiler_params=_SC_PARAMS,
)


def _spmm_body(W, BT, src_hbm, dst_hbm, norm_hbm, x_hbm, out_hbm,
               acc, sbuf, dbuf, nbuf, csrc, coff, cnrm, rbuf):
    c = lax.axis_index("c")
    s = lax.axis_index("s")
    w = c * 16 + s
    z16 = jnp.zeros((16,), F32)
    z16i = jnp.zeros((16,), I32)
    wlanes = W // 16

    def _pass(p, _):
        rbase = p * 5120 + w * 160

        def _zb(i, _):
            acc[pl.ds(i * 16, 16)] = z16
            return 0
        lax.fori_loop(0, 160 * wlanes, _zb, 0)

        def _batch(b, _):
            bbase = b * BT
            pltpu.sync_copy(src_hbm.at[pl.ds(bbase, BT)], sbuf)
            pltpu.sync_copy(dst_hbm.at[pl.ds(bbase, BT)], dbuf)
            pltpu.sync_copy(norm_hbm.at[pl.ds(bbase, BT)], nbuf)

            def _cb(i, off):
                d16 = dbuf[pl.ds(i * 16, 16)]
                m = (d16 >= rbase) & (d16 < rbase + 160)
                plsc.store_compressed(csrc.at[pl.ds(off, 16)],
                                      sbuf[pl.ds(i * 16, 16)], mask=m)
                plsc.store_compressed(coff.at[pl.ds(off, 16)], d16 - rbase,
                                      mask=m)
                plsc.store_compressed(cnrm.at[pl.ds(off, 16)],
                                      nbuf[pl.ds(i * 16, 16)], mask=m)
                return off + jnp.sum(m.astype(I32))
            cnt = lax.fori_loop(0, BT // 16, _cb, jnp.int32(0))

            csrc[pl.ds(cnt, 16)] = z16i
            coff[pl.ds(cnt, 16)] = z16i
            cnrm[pl.ds(cnt, 16)] = z16
            nch = (cnt + 15) // 16

            def _gb(ch, _):
                i16 = csrc[pl.ds(ch * 16, 16)]
                pltpu.sync_copy(x_hbm.at[i16], rbuf)
                n16 = cnrm[pl.ds(ch * 16, 16)]
                o16 = coff[pl.ds(ch * 16, 16)]
                for r in range(16):
                    n = n16[r]
                    o = o16[r] * W

                    def _jb(j, _):
                        q = o + j * 16
                        acc[pl.ds(q, 16)] = (acc[pl.ds(q, 16)]
                                             + n * rbuf[r, pl.ds(j * 16, 16)])
                        return 0
                    lax.fori_loop(0, wlanes, _jb, 0)
                return 0
            lax.fori_loop(0, nch, _gb, 0)
            return 0
        lax.fori_loop(0, EPAD // BT, _batch, 0)

        pltpu.sync_copy(acc, out_hbm.at[pl.ds(rbase * W, 160 * W)])
        return 0
    lax.fori_loop(0, 2, _pass, 0)


def _make_spmm(W, BT):
    return pl.kernel(
        functools.partial(_spmm_body, W, BT),
        out_type=jax.ShapeDtypeStruct((NPAD * W,), F32),
        mesh=_MESH,
        scratch_types=[
            pltpu.VMEM((160 * W,), F32),
            pltpu.VMEM((BT,), I32),
            pltpu.VMEM((BT,), I32),
            pltpu.VMEM((BT,), F32),
            pltpu.VMEM((BT + 128,), I32),
            pltpu.VMEM((BT + 128,), I32),
            pltpu.VMEM((BT + 128,), F32),
            pltpu.VMEM((16, W), F32),
        ],
        compiler_params=_SC_PARAMS,
    )


_spmm640 = _make_spmm(640, 2048)
_spmm512 = _make_spmm(512, 4096)


def _gcn_mm_body(t_ref, w1_ref, w2_ref, b2_ref, out_ref):
    h = jnp.dot(t_ref[...], w1_ref[...], preferred_element_type=F32)
    h = jnp.maximum(h, 0.0)
    out_ref[...] = (jnp.dot(h, w2_ref[...], preferred_element_type=F32)
                    + b2_ref[...])


def _gcn_mm(t_aug, w1_aug, w2, b2):
    return pl.pallas_call(
        _gcn_mm_body,
        grid=(NPAD // 256,),
        in_specs=[
            pl.BlockSpec((256, 640), lambda i: (i, 0)),
            pl.BlockSpec((640, 4096), lambda i: (0, 0)),
            pl.BlockSpec((4096, 512), lambda i: (0, 0)),
            pl.BlockSpec((1, 512), lambda i: (0, 0)),
        ],
        out_specs=pl.BlockSpec((256, 512), lambda i: (i, 0)),
        out_shape=jax.ShapeDtypeStruct((NPAD, 512), F32),
    )(t_aug, w1_aug, w2, b2)


def _score_body(img_ref, wi1_ref, bi1_ref, wi2_ref, bi2_ref, emb_ref,
                out_ref, feats_scr):
    @pl.when(pl.program_id(0) == 0)
    def _():
        f = jnp.dot(img_ref[...], wi1_ref[...], preferred_element_type=F32)
        f = jnp.maximum(f + bi1_ref[...], 0.0)
        feats_scr[...] = (jnp.dot(f, wi2_ref[...], preferred_element_type=F32)
                          + bi2_ref[...])

    out_ref[...] = lax.dot_general(
        feats_scr[...], emb_ref[...],
        dimension_numbers=(((1,), (1,)), ((), ())),
        preferred_element_type=F32)


def _score_mm(img, wi1, bi1, wi2, bi2, emb_pairs):
    return pl.pallas_call(
        _score_body,
        grid=(9216 // 256,),
        in_specs=[
            pl.BlockSpec((1024, 512), lambda i: (0, 0)),
            pl.BlockSpec((512, 768), lambda i: (0, 0)),
            pl.BlockSpec((1, 768), lambda i: (0, 0)),
            pl.BlockSpec((768, 512), lambda i: (0, 0)),
            pl.BlockSpec((1, 512), lambda i: (0, 0)),
            pl.BlockSpec((256, 512), lambda i: (i, 0)),
        ],
        out_specs=pl.BlockSpec((1024, 256), lambda i: (0, i)),
        out_shape=jax.ShapeDtypeStruct((1024, 9216), F32),
        scratch_shapes=[pltpu.VMEM((1024, 512), F32)],
    )(img, wi1, bi1, wi2, bi2, emb_pairs)


def kernel(x, edge_index, edge_weight, img, W1, b1, W2, b2,
           Wi1, bi1, Wi2, bi2):
    src = edge_index[0].astype(I32)
    dst = edge_index[1].astype(I32)
    ew = edge_weight.astype(F32)
    pad = EPAD - E
    src = jnp.pad(src, (0, pad))
    dst = jnp.pad(dst, (0, pad))
    ew = jnp.pad(ew, (0, pad))

    deg_parts = _deg_kernel(dst, ew)
    dinv = _dinv_tc(deg_parts)
    norm = _norm_kernel(src, dst, ew, dinv)

    n = x.shape[0]
    x_aug = jnp.concatenate(
        [x, jnp.ones((n, 1), F32), jnp.zeros((n, 39), F32)], axis=1)
    t_aug = _spmm640(src, dst, norm, x_aug).reshape(NPAD, 640)

    w1_aug = jnp.concatenate(
        [W1, b1[None, :], jnp.zeros((39, 4096), F32)], axis=0)
    g = _gcn_mm(t_aug, w1_aug, W2, b2[None, :])

    emb = _spmm512(src, dst, norm, g).reshape(NPAD, 512)
    emb_pairs = emb[1000:10216]
    score = _score_mm(img, Wi1, bi1[None, :], Wi2, bi2[None, :], emb_pairs)
    return score[:, :9000]


# speedup vs baseline: 1.6241x; 1.0344x over previous
import functools

import jax
import jax.numpy as jnp
from jax import lax
from jax.experimental import pallas as pl
from jax.experimental.pallas import tpu as pltpu
from jax.experimental.pallas import tpu_sc as plsc

N = 10000
NPAD = 10240
E = 160000
EPAD = 163840
NUM_RANGES = 4
RROWS = NPAD // NUM_RANGES
F32 = jnp.float32
I32 = jnp.int32

_MESH = plsc.VectorSubcoreMesh(core_axis_name="c", subcore_axis_name="s")
_SC_PARAMS = pltpu.CompilerParams(needs_layout_passes=False)


def _iota16():
    return lax.iota(I32, 16)


def _deg_body(dst_hbm, ew_hbm, deg_hbm, dacc, dloc, ewloc):
    c = lax.axis_index("c")
    s = lax.axis_index("s")
    w = c * 16 + s
    z16 = jnp.zeros((16,), F32)

    def _zb(i, _):
        dacc[pl.ds(i * 16, 16)] = z16
        return 0
    lax.fori_loop(0, NPAD // 16, _zb, 0)

    base = w * 5120
    pltpu.sync_copy(dst_hbm.at[pl.ds(base, 5120)], dloc)
    pltpu.sync_copy(ew_hbm.at[pl.ds(base, 5120)], ewloc)

    def _eb(i, _):
        idx = dloc[pl.ds(i * 16, 16)]
        plsc.addupdate_scatter(dacc, [idx], ewloc[pl.ds(i * 16, 16)])
        return 0
    lax.fori_loop(0, 320, _eb, 0)
    pltpu.sync_copy(dacc, deg_hbm.at[pl.ds(w * NPAD, NPAD)])


_deg_kernel = pl.kernel(
    _deg_body,
    out_type=jax.ShapeDtypeStruct((32 * NPAD,), F32),
    mesh=_MESH,
    scratch_types=[
        pltpu.VMEM((NPAD,), F32),
        pltpu.VMEM((5120,), I32),
        pltpu.VMEM((5120,), F32),
    ],
    compiler_params=_SC_PARAMS,
)


def _dinv_body(deg_ref, out_ref):
    d = deg_ref[pl.ds(0, NPAD // 128), :]
    for k in range(1, 32):
        d = d + deg_ref[pl.ds(k * (NPAD // 128), NPAD // 128), :]
    out_ref[...] = lax.rsqrt(jnp.maximum(d, 1e-6))


def _dinv_tc(deg_parts):
    return pl.pallas_call(
        _dinv_body,
        out_shape=jax.ShapeDtypeStruct((NPAD // 128, 128), F32),
    )(deg_parts.reshape(32 * (NPAD // 128), 128)).reshape(NPAD)


def _norm_body(src_hbm, dst_hbm, ew_hbm, dinv_hbm, norm_hbm,
               dfull, sloc, dloc, ewloc, nout):
    c = lax.axis_index("c")
    s = lax.axis_index("s")
    w = c * 16 + s

    pltpu.sync_copy(dinv_hbm, dfull)
    base = w * 5120
    pltpu.sync_copy(src_hbm.at[pl.ds(base, 5120)], sloc)
    pltpu.sync_copy(dst_hbm.at[pl.ds(base, 5120)], dloc)
    pltpu.sync_copy(ew_hbm.at[pl.ds(base, 5120)], ewloc)

    def _nb(i, _):
        s16 = sloc[pl.ds(i * 16, 16)]
        d16 = dloc[pl.ds(i * 16, 16)]
        dv_s = plsc.load_gather(dfull, [s16])
        dv_d = plsc.load_gather(dfull, [d16])
        nout[pl.ds(i * 16, 16)] = ewloc[pl.ds(i * 16, 16)] * dv_s * dv_d
        return 0
    lax.fori_loop(0, 320, _nb, 0)
    pltpu.sync_copy(nout, norm_hbm.at[pl.ds(base, 5120)])


_norm_kernel = pl.kernel(
    _norm_body,
    out_type=jax.ShapeDtypeStruct((EPAD,), F32),
    mesh=_MESH,
    scratch_types=[
        pltpu.VMEM((NPAD,), F32),
        pltpu.VMEM((5120,), I32),
        pltpu.VMEM((5120,), I32),
        pltpu.VMEM((5120,), F32),
        pltpu.VMEM((5120,), F32),
    ],
    compiler_params=_SC_PARAMS,
)


def _spmm_body(W, BT, src_hbm, dst_hbm, norm_hbm, x_hbm, out_hbm,
               acc, sbuf, dbuf, nbuf, csrc, coff, cnrm, rbuf):
    c = lax.axis_index("c")
    s = lax.axis_index("s")
    w = c * 16 + s
    z16 = jnp.zeros((16,), F32)
    z16i = jnp.zeros((16,), I32)
    wlanes = W // 16

    def _pass(p, _):
        rbase = p * 5120 + w * 160

        def _zb(i, _):
            acc[pl.ds(i * 16, 16)] = z16
            return 0
        lax.fori_loop(0, 160 * wlanes, _zb, 0)

        def _batch(b, _):
            bbase = b * BT
            pltpu.sync_copy(src_hbm.at[pl.ds(bbase, BT)], sbuf)
            pltpu.sync_copy(dst_hbm.at[pl.ds(bbase, BT)], dbuf)
            pltpu.sync_copy(norm_hbm.at[pl.ds(bbase, BT)], nbuf)

            def _cb(i, off):
                d16 = dbuf[pl.ds(i * 16, 16)]
                m = (d16 >= rbase) & (d16 < rbase + 160)
                plsc.store_compressed(csrc.at[pl.ds(off, 16)],
                                      sbuf[pl.ds(i * 16, 16)], mask=m)
                plsc.store_compressed(coff.at[pl.ds(off, 16)], d16 - rbase,
                                      mask=m)
                plsc.store_compressed(cnrm.at[pl.ds(off, 16)],
                                      nbuf[pl.ds(i * 16, 16)], mask=m)
                return off + jnp.sum(m.astype(I32))
            cnt = lax.fori_loop(0, BT // 16, _cb, jnp.int32(0))

            csrc[pl.ds(cnt, 16)] = z16i
            coff[pl.ds(cnt, 16)] = z16i
            cnrm[pl.ds(cnt, 16)] = z16
            nch = (cnt + 15) // 16

            def _gb(ch, _):
                i16 = csrc[pl.ds(ch * 16, 16)]
                pltpu.sync_copy(x_hbm.at[i16], rbuf)
                n16 = cnrm[pl.ds(ch * 16, 16)]
                o16 = coff[pl.ds(ch * 16, 16)]
                for r in range(16):
                    n = n16[r]
                    o = o16[r] * W

                    def _jb(j, _):
                        q = o + j * 16
                        acc[pl.ds(q, 16)] = (acc[pl.ds(q, 16)]
                                             + n * rbuf[r, pl.ds(j * 16, 16)])
                        return 0
                    lax.fori_loop(0, wlanes, _jb, 0)
                return 0
            lax.fori_loop(0, nch, _gb, 0)
            return 0
        lax.fori_loop(0, EPAD // BT, _batch, 0)

        pltpu.sync_copy(acc, out_hbm.at[pl.ds(rbase * W, 160 * W)])
        return 0
    lax.fori_loop(0, 2, _pass, 0)


def _make_spmm(W, BT):
    return pl.kernel(
        functools.partial(_spmm_body, W, BT),
        out_type=jax.ShapeDtypeStruct((NPAD * W,), F32),
        mesh=_MESH,
        scratch_types=[
            pltpu.VMEM((160 * W,), F32),
            pltpu.VMEM((BT,), I32),
            pltpu.VMEM((BT,), I32),
            pltpu.VMEM((BT,), F32),
            pltpu.VMEM((BT + 128,), I32),
            pltpu.VMEM((BT + 128,), I32),
            pltpu.VMEM((BT + 128,), F32),
            pltpu.VMEM((16, W), F32),
        ],
        compiler_params=_SC_PARAMS,
    )


_spmm640 = _make_spmm(640, 2560)
_spmm512 = _make_spmm(512, 5120)


def _gcn_mm_body(t_ref, w1_ref, w2_ref, b2_ref, out_ref):
    h = jnp.dot(t_ref[...], w1_ref[...], preferred_element_type=F32)
    h = jnp.maximum(h, 0.0)
    out_ref[...] = (jnp.dot(h, w2_ref[...], preferred_element_type=F32)
                    + b2_ref[...])


def _gcn_mm(t_aug, w1_aug, w2, b2):
    return pl.pallas_call(
        _gcn_mm_body,
        grid=(NPAD // 256,),
        in_specs=[
            pl.BlockSpec((256, 640), lambda i: (i, 0)),
            pl.BlockSpec((640, 4096), lambda i: (0, 0)),
            pl.BlockSpec((4096, 512), lambda i: (0, 0)),
            pl.BlockSpec((1, 512), lambda i: (0, 0)),
        ],
        out_specs=pl.BlockSpec((256, 512), lambda i: (i, 0)),
        out_shape=jax.ShapeDtypeStruct((NPAD, 512), F32),
    )(t_aug, w1_aug, w2, b2)


def _score_body(img_ref, wi1_ref, bi1_ref, wi2_ref, bi2_ref, emb_ref,
                out_ref, feats_scr):
    @pl.when(pl.program_id(0) == 0)
    def _():
        f = jnp.dot(img_ref[...], wi1_ref[...], preferred_element_type=F32)
        f = jnp.maximum(f + bi1_ref[...], 0.0)
        feats_scr[...] = (jnp.dot(f, wi2_ref[...], preferred_element_type=F32)
                          + bi2_ref[...])

    out_ref[...] = lax.dot_general(
        feats_scr[...], emb_ref[...],
        dimension_numbers=(((1,), (1,)), ((), ())),
        preferred_element_type=F32)


def _score_mm(img, wi1, bi1, wi2, bi2, emb_pairs):
    return pl.pallas_call(
        _score_body,
        grid=(9216 // 256,),
        in_specs=[
            pl.BlockSpec((1024, 512), lambda i: (0, 0)),
            pl.BlockSpec((512, 768), lambda i: (0, 0)),
            pl.BlockSpec((1, 768), lambda i: (0, 0)),
            pl.BlockSpec((768, 512), lambda i: (0, 0)),
            pl.BlockSpec((1, 512), lambda i: (0, 0)),
            pl.BlockSpec((256, 512), lambda i: (i, 0)),
        ],
        out_specs=pl.BlockSpec((1024, 256), lambda i: (0, i)),
        out_shape=jax.ShapeDtypeStruct((1024, 9216), F32),
        scratch_shapes=[pltpu.VMEM((1024, 512), F32)],
    )(img, wi1, bi1, wi2, bi2, emb_pairs)


def kernel(x, edge_index, edge_weight, img, W1, b1, W2, b2,
           Wi1, bi1, Wi2, bi2):
    src = edge_index[0].astype(I32)
    dst = edge_index[1].astype(I32)
    ew = edge_weight.astype(F32)
    pad = EPAD - E
    src = jnp.pad(src, (0, pad))
    dst = jnp.pad(dst, (0, pad))
    ew = jnp.pad(ew, (0, pad))

    deg_parts = _deg_kernel(dst, ew)
    dinv = _dinv_tc(deg_parts)
    norm = _norm_kernel(src, dst, ew, dinv)

    n = x.shape[0]
    x_aug = jnp.concatenate(
        [x, jnp.ones((n, 1), F32), jnp.zeros((n, 39), F32)], axis=1)
    t_aug = _spmm640(src, dst, norm, x_aug).reshape(NPAD, 640)

    w1_aug = jnp.concatenate(
        [W1, b1[None, :], jnp.zeros((39, 4096), F32)], axis=0)
    g = _gcn_mm(t_aug, w1_aug, W2, b2[None, :])

    emb = _spmm512(src, dst, norm, g).reshape(NPAD, 512)
    emb_pairs = emb[1000:10216]
    score = _score_mm(img, Wi1, bi1[None, :], Wi2, bi2[None, :], emb_pairs)
    return score[:, :9000]
